# baseline (speedup 1.0000x reference)
import jax
import jax.numpy as jnp
from jax import lax
from jax.experimental import pallas as pl
from jax.experimental.pallas import tpu as pltpu

_N, _E, _D, _DE = 10000, 320000, 128, 16
_PACK = _D // _DE
_EP = _E // _PACK
_G = 25
_NBLK = _N // _G
_EBLK = _EP // _G


def _bce(logits, labels):
    return (jnp.maximum(logits, 0.0) - logits * labels
            + jnp.log1p(jnp.exp(-jnp.abs(logits))))


def _fused(nf_ref, nlab_ref, ef_ref, elab_ref,
           wn1_ref, bn1_ref, wn2t_ref, bn2_ref,
           k1_ref, b1t_ref, k2_ref, be2_ref,
           dn_ref, de_ref,
           nlog_ref, elog_ref, nsum_ref, esum_ref):
    i = pl.program_id(0)

    nh = jnp.maximum(
        jnp.dot(nf_ref[...], wn1_ref[...], preferred_element_type=jnp.float32)
        + bn1_ref[...], 0.0)
    nlogit = (lax.dot_general(wn2t_ref[...], nh, (((1,), (1,)), ((), ())),
                              preferred_element_type=jnp.float32)
              + bn2_ref[...])
    nlog_ref[...] = nlogit.reshape(1, 1, _NBLK)

    eh = jnp.maximum(
        jnp.dot(ef_ref[...], k1_ref[...], preferred_element_type=jnp.float32)
        + b1t_ref[...], 0.0)
    elogit = (lax.dot_general(k2_ref[...], eh, (((1,), (1,)), ((), ())),
                              preferred_element_type=jnp.float32)
              + be2_ref[...])
    elog_ref[...] = elogit.reshape(1, _PACK, _EBLK)

    @pl.when(i == 0)
    def _init():
        nsum_ref[...] = jnp.zeros_like(nsum_ref)
        esum_ref[...] = jnp.zeros_like(esum_ref)

    nlab = nlab_ref[...].reshape(1, _NBLK)
    elab = elab_ref[...].reshape(_PACK, _EBLK)
    nsum_ref[...] += jnp.sum(_bce(nlogit, nlab)).reshape(1, 1)
    esum_ref[...] += jnp.sum(_bce(elogit, elab)).reshape(1, 1)

    @pl.when(i == _G - 1)
    def _norm():
        nsum_ref[...] = nsum_ref[...] / dn_ref[...]
        esum_ref[...] = esum_ref[...] / de_ref[...]


def kernel(node_feat, edge_feat, node_label, edge_label, node_batch,
           edge_batch, Wn1, bn1, Wn2, bn2, We1, be1, We2, be2):
    ef = edge_feat.reshape(_EP, _D)
    elabT = edge_label.reshape(_G, _EBLK, _PACK).transpose(0, 2, 1)
    nlab = node_label.reshape(_G, 1, _NBLK)

    eye = jnp.eye(_PACK, dtype=We1.dtype)
    K1 = jnp.kron(eye, We1)
    b1t = jnp.tile(be1, _PACK).reshape(1, _D)
    K2 = jnp.kron(eye, We2.T)

    dn = (node_batch[-1].astype(jnp.float32) + 1.0).reshape(1, 1)
    de = (edge_batch[-1].astype(jnp.float32) + 1.0).reshape(1, 1)

    row = lambda i: (i, 0)
    row3 = lambda i: (i, 0, 0)
    fixed = lambda i: (0, 0)
    full = lambda a: pl.BlockSpec(a.shape, fixed)

    nlog, elogT, nsum, esum = pl.pallas_call(
        _fused,
        grid=(_G,),
        in_specs=[
            pl.BlockSpec((_NBLK, _D), row),
            pl.BlockSpec((1, 1, _NBLK), row3),
            pl.BlockSpec((_EBLK, _D), row),
            pl.BlockSpec((1, _PACK, _EBLK), row3),
            full(Wn1),
            pl.BlockSpec((1, _D), fixed),
            pl.BlockSpec((1, _D), fixed),
            pl.BlockSpec((1, 1), fixed),
            pl.BlockSpec((_D, _D), fixed),
            pl.BlockSpec((1, _D), fixed),
            pl.BlockSpec((_PACK, _D), fixed),
            pl.BlockSpec((1, 1), fixed),
            pl.BlockSpec((1, 1), fixed),
            pl.BlockSpec((1, 1), fixed),
        ],
        out_specs=[
            pl.BlockSpec((1, 1, _NBLK), row3),
            pl.BlockSpec((1, _PACK, _EBLK), row3),
            pl.BlockSpec((1, 1), fixed),
            pl.BlockSpec((1, 1), fixed),
        ],
        out_shape=[
            jax.ShapeDtypeStruct((_G, 1, _NBLK), jnp.float32),
            jax.ShapeDtypeStruct((_G, _PACK, _EBLK), jnp.float32),
            jax.ShapeDtypeStruct((1, 1), jnp.float32),
            jax.ShapeDtypeStruct((1, 1), jnp.float32),
        ],
        compiler_params=pltpu.CompilerParams(
            dimension_semantics=("arbitrary",)),
    )(node_feat, nlab, ef, elabT,
      Wn1, bn1.reshape(1, _D), Wn2.T, bn2.reshape(1, 1),
      K1, b1t, K2, be2.reshape(1, 1),
      dn, de)

    elog = elogT.transpose(0, 2, 1).reshape(_E)
    return (nlog.reshape(_N), elog, nsum[0, 0], esum[0, 0])


# speedup vs baseline: 4.7782x; 4.7782x over previous
import jax
import jax.numpy as jnp
from jax import lax
from jax.experimental import pallas as pl
from jax.experimental.pallas import tpu as pltpu

_N, _E, _D, _DE = 10000, 320000, 128, 16
_G = 25
_NBLK = _N // _G
_EBLK = _E // _G
_S = 8
_ECH = _EBLK // _S


def _bce(logits, labels):
    return (jnp.maximum(logits, 0.0) - logits * labels
            + jnp.log1p(jnp.exp(-jnp.abs(logits))))


def _fused(nf_ref, nlab_ref, eft_ref, elab_ref,
           wn1_ref, bn1_ref, wn2t_ref, bn2_ref,
           we1t_ref, be1_ref, we2t_ref, be2_ref,
           dn_ref, de_ref,
           nlog_ref, elog_ref, nsum_ref, esum_ref):
    i = pl.program_id(0)

    nh = jnp.maximum(
        jnp.dot(nf_ref[...], wn1_ref[...], preferred_element_type=jnp.float32)
        + bn1_ref[...], 0.0)
    nlogit = (lax.dot_general(wn2t_ref[...], nh, (((1,), (1,)), ((), ())),
                              preferred_element_type=jnp.float32)
              + bn2_ref[...])
    nlog_ref[...] = nlogit.reshape(1, 1, _NBLK)

    eh = jnp.maximum(
        jnp.dot(we1t_ref[...], eft_ref[...], preferred_element_type=jnp.float32)
        + be1_ref[...], 0.0)
    w2 = we2t_ref[...]
    elogit = jnp.concatenate(
        [jnp.dot(w2, eh[:, s * _ECH:(s + 1) * _ECH],
                 preferred_element_type=jnp.float32) for s in range(_S)],
        axis=0) + be2_ref[...]
    elog_ref[...] = elogit.reshape(1, _S, _ECH)

    @pl.when(i == 0)
    def _init():
        nsum_ref[...] = jnp.zeros_like(nsum_ref)
        esum_ref[...] = jnp.zeros_like(esum_ref)

    nlab = nlab_ref[...].reshape(1, _NBLK)
    elab = elab_ref[...].reshape(_S, _ECH)
    nsum_ref[...] += jnp.sum(_bce(nlogit, nlab)).reshape(1, 1)
    esum_ref[...] += jnp.sum(_bce(elogit, elab)).reshape(1, 1)

    @pl.when(i == _G - 1)
    def _norm():
        nsum_ref[...] = nsum_ref[...] / dn_ref[...]
        esum_ref[...] = esum_ref[...] / de_ref[...]


def kernel(node_feat, edge_feat, node_label, edge_label, node_batch,
           edge_batch, Wn1, bn1, Wn2, bn2, We1, be1, We2, be2):
    eft = edge_feat.T
    elab = edge_label.reshape(_G, _S, _ECH)
    nlab = node_label.reshape(_G, 1, _NBLK)

    dn = (node_batch[-1].astype(jnp.float32) + 1.0).reshape(1, 1)
    de = (edge_batch[-1].astype(jnp.float32) + 1.0).reshape(1, 1)

    row3 = lambda i: (i, 0, 0)
    col = lambda i: (0, i)
    fixed = lambda i: (0, 0)
    full = lambda a: pl.BlockSpec(a.shape, fixed)

    nlog, elog, nsum, esum = pl.pallas_call(
        _fused,
        grid=(_G,),
        in_specs=[
            pl.BlockSpec((_NBLK, _D), lambda i: (i, 0)),
            pl.BlockSpec((1, 1, _NBLK), row3),
            pl.BlockSpec((_DE, _EBLK), col),
            pl.BlockSpec((1, _S, _ECH), row3),
            full(Wn1),
            pl.BlockSpec((1, _D), fixed),
            pl.BlockSpec((1, _D), fixed),
            pl.BlockSpec((1, 1), fixed),
            pl.BlockSpec((_DE, _DE), fixed),
            pl.BlockSpec((_DE, 1), fixed),
            pl.BlockSpec((1, _DE), fixed),
            pl.BlockSpec((1, 1), fixed),
            pl.BlockSpec((1, 1), fixed),
            pl.BlockSpec((1, 1), fixed),
        ],
        out_specs=[
            pl.BlockSpec((1, 1, _NBLK), row3),
            pl.BlockSpec((1, _S, _ECH), row3),
            pl.BlockSpec((1, 1), fixed),
            pl.BlockSpec((1, 1), fixed),
        ],
        out_shape=[
            jax.ShapeDtypeStruct((_G, 1, _NBLK), jnp.float32),
            jax.ShapeDtypeStruct((_G, _S, _ECH), jnp.float32),
            jax.ShapeDtypeStruct((1, 1), jnp.float32),
            jax.ShapeDtypeStruct((1, 1), jnp.float32),
        ],
        compiler_params=pltpu.CompilerParams(
            dimension_semantics=("arbitrary",)),
    )(node_feat, nlab, eft, elab,
      Wn1, bn1.reshape(1, _D), Wn2.T, bn2.reshape(1, 1),
      We1.T, be1.reshape(_DE, 1), We2.T, be2.reshape(1, 1),
      dn, de)

    return (nlog.reshape(_N), elog.reshape(_E), nsum[0, 0], esum[0, 0])
